# Initial kernel scaffold; baseline (speedup 1.0000x reference)
import jax
import jax.numpy as jnp
from jax.experimental import pallas as pl


def kernel(x, enc_weight):
    raise NotImplementedError("write your pallas kernel here")


# speedup vs baseline: 3.3787x; 3.3787x over previous
import jax
import jax.numpy as jnp
from jax.experimental import pallas as pl
from jax.experimental.pallas import tpu as pltpu

_BLOCK_ROWS = 512


def _copy_block_kernel(w_ref, o_ref):
    o_ref[...] = w_ref[...]


def kernel(x, enc_weight):
    seq_len = x.shape[1]
    d = enc_weight.shape[1]
    out_shape = jax.ShapeDtypeStruct((seq_len, d), enc_weight.dtype)
    grid = (seq_len // _BLOCK_ROWS,)
    return pl.pallas_call(
        _copy_block_kernel,
        out_shape=out_shape,
        grid=grid,
        in_specs=[pl.BlockSpec((_BLOCK_ROWS, d), lambda i: (i, 0))],
        out_specs=pl.BlockSpec((_BLOCK_ROWS, d), lambda i: (i, 0)),
    )(enc_weight)
